# Initial kernel scaffold; baseline (speedup 1.0000x reference)
import jax
import jax.numpy as jnp
from jax.experimental import pallas as pl


def kernel(x, edge_index, W1, b1, W2, b2, W3, b3, W4, b4, W5, b5):
    raise NotImplementedError("write your pallas kernel here")


# speedup vs baseline: 14.8371x; 14.8371x over previous
import functools

import jax
import jax.numpy as jnp
from jax import lax
from jax.experimental import pallas as pl
from jax.experimental.pallas import tpu as pltpu
from jax.experimental.pallas import tpu_sc as plsc

N = 10000
HID = 64
NC, NS, L = 2, 16, 16
NW = NC * NS
NPAD = 10240
CH = 128
ROWS_PER_TILE = NPAD // NS


def _mesh():
    return plsc.VectorSubcoreMesh(core_axis_name="c", subcore_axis_name="s")


@functools.partial(jax.jit, static_argnums=(1,))
def _sc_hist(dst_pad, e_pad):
    epw = e_pad // NW
    n_chunks = epw // CH

    @functools.partial(
        pl.kernel,
        out_type=jax.ShapeDtypeStruct((NW * NPAD,), jnp.float32),
        mesh=_mesh(),
        scratch_types=[
            pltpu.VMEM((NPAD,), jnp.float32),
            pltpu.VMEM((CH,), jnp.int32),
        ],
        compiler_params=pltpu.CompilerParams(needs_layout_passes=False),
    )
    def hist_k(dst_hbm, out_hbm, hist_v, idx_v):
        cid = lax.axis_index("c")
        sid = lax.axis_index("s")
        wid = cid * NS + sid

        def zbody(i, carry):
            hist_v[pl.ds(i * L, L)] = jnp.zeros((L,), jnp.float32)
            return carry

        lax.fori_loop(0, NPAD // L, zbody, 0)

        ones = jnp.ones((L,), jnp.float32)

        def cbody(k, carry):
            base = wid * epw + k * CH
            pltpu.sync_copy(dst_hbm.at[pl.ds(base, CH)], idx_v)

            def ibody(j, c2):
                idx = idx_v[pl.ds(j * L, L)]
                plsc.addupdate_scatter(hist_v, [idx], ones)
                return c2

            lax.fori_loop(0, CH // L, ibody, 0)
            return carry

        lax.fori_loop(0, n_chunks, cbody, 0)
        pltpu.sync_copy(hist_v, out_hbm.at[pl.ds(wid * NPAD, NPAD)])

    return hist_k(dst_pad).reshape(NW, NPAD)


@functools.partial(jax.jit, static_argnums=(3,))
def _sc_scatter(u_pad, src_pad, dst_pad, e_pad):
    epw = e_pad // NW
    n_chunks = epw // CH

    @functools.partial(
        pl.kernel,
        out_type=jax.ShapeDtypeStruct((NC * NPAD, HID), jnp.float32),
        mesh=_mesh(),
        scratch_types=[
            pltpu.VMEM_SHARED((NPAD, HID), jnp.float32),
            pltpu.VMEM((CH,), jnp.int32),
            pltpu.VMEM((CH,), jnp.int32),
            pltpu.VMEM((CH, HID), jnp.float32),
            pltpu.SemaphoreType.DMA,
        ],
        compiler_params=pltpu.CompilerParams(use_tc_tiling_on_sc=False),
    )
    def scat_k(u_hbm, src_hbm, dst_hbm, out_hbm, acc_sh, idx_s, idx_d,
               rows_v, sem):
        cid = lax.axis_index("c")
        sid = lax.axis_index("s")
        wid = cid * NS + sid

        def zb(i, carry):
            rows_v[i, pl.ds(0, L)] = jnp.zeros((L,), jnp.float32)
            rows_v[i, pl.ds(L, L)] = jnp.zeros((L,), jnp.float32)
            rows_v[i, pl.ds(2 * L, L)] = jnp.zeros((L,), jnp.float32)
            rows_v[i, pl.ds(3 * L, L)] = jnp.zeros((L,), jnp.float32)
            return carry

        lax.fori_loop(0, CH, zb, 0)

        def zcopy(t, carry):
            pltpu.sync_copy(rows_v,
                            acc_sh.at[pl.ds(sid * ROWS_PER_TILE + t * CH, CH)])
            return carry

        lax.fori_loop(0, ROWS_PER_TILE // CH, zcopy, 0)
        plsc.subcore_barrier()

        def cbody(k, carry):
            base = wid * epw + k * CH
            pltpu.sync_copy(src_hbm.at[pl.ds(base, CH)], idx_s)
            pltpu.sync_copy(dst_hbm.at[pl.ds(base, CH)], idx_d)
            pltpu.async_copy(u_hbm.at[idx_s], rows_v, sem).wait()
            pltpu.sync_copy(rows_v, acc_sh.at[idx_d], add=True)
            return carry

        lax.fori_loop(0, n_chunks, cbody, 0)
        plsc.subcore_barrier()
        pltpu.sync_copy(
            acc_sh.at[pl.ds(sid * ROWS_PER_TILE, ROWS_PER_TILE)],
            out_hbm.at[pl.ds(cid * NPAD + sid * ROWS_PER_TILE,
                             ROWS_PER_TILE)])

    return scat_k(u_pad, src_pad, dst_pad).reshape(NC, NPAD, HID)


def _tc_prep(x, W1, hist_t):
    def body(x_ref, w_ref, h_ref, u_ref, dinv_ref):
        deg = jnp.sum(h_ref[...], axis=1, keepdims=True) + 1.0
        dinv = lax.rsqrt(deg)[:N, :]
        h = jnp.dot(x_ref[...], w_ref[...],
                    preferred_element_type=jnp.float32)
        u_ref[...] = h * dinv
        dinv_ref[...] = dinv

    return pl.pallas_call(
        body,
        out_shape=[jax.ShapeDtypeStruct((N, HID), jnp.float32),
                   jax.ShapeDtypeStruct((N, 1), jnp.float32)],
    )(x, W1, hist_t)


def _tc_mid(p0, p1, u0, dinv, b1, W2):
    def body(p0_ref, p1_ref, u_ref, dinv_ref, b_ref, w_ref, out_ref):
        dinv = dinv_ref[...]
        h0 = dinv * (p0_ref[...] + p1_ref[...] + u_ref[...]) + b_ref[...]
        h0 = jnp.maximum(h0, 0.0)
        out_ref[...] = dinv * jnp.dot(h0, w_ref[...],
                                      preferred_element_type=jnp.float32)

    return pl.pallas_call(
        body,
        out_shape=jax.ShapeDtypeStruct((N, HID), jnp.float32),
    )(p0, p1, u0, dinv, b1, W2)


def _tc_final(p0, p1, u1, dinv, b2, W3, b3, W4, b4, W5, b5):
    def body(p0_ref, p1_ref, u_ref, dinv_ref, b2_ref, w3_ref, b3_ref,
             w4_ref, b4_ref, w5_ref, b5_ref, out_ref):
        h1 = dinv_ref[...] * (p0_ref[...] + p1_ref[...] + u_ref[...]) \
            + b2_ref[...]
        mean = jnp.mean(h1, axis=0, keepdims=True)
        mx = jnp.max(h1, axis=0, keepdims=True)
        logits = jnp.sum(h1 * mean, axis=1, keepdims=True)
        m = jnp.max(logits, axis=0, keepdims=True)
        e = jnp.exp(logits - m)
        att = e / jnp.sum(e, axis=0, keepdims=True)
        attp = jnp.sum(h1 * att, axis=0, keepdims=True)
        comb = jnp.concatenate([mean, mx, attp], axis=1)
        g = jnp.maximum(
            jnp.dot(comb, w3_ref[...], preferred_element_type=jnp.float32)
            + b3_ref[...], 0.0)
        g = jnp.maximum(
            jnp.dot(g, w4_ref[...], preferred_element_type=jnp.float32)
            + b4_ref[...], 0.0)
        out_ref[...] = jnp.dot(g, w5_ref[...],
                               preferred_element_type=jnp.float32) + b5_ref[...]

    return pl.pallas_call(
        body,
        out_shape=jax.ShapeDtypeStruct((1, 128), jnp.float32),
    )(p0, p1, u1, dinv, b2, W3, b3, W4, b4, W5, b5)


def kernel(x, edge_index, W1, b1, W2, b2, W3, b3, W4, b4, W5, b5):
    E = edge_index.shape[1]
    e_pad = ((E + NW * CH - 1) // (NW * CH)) * (NW * CH)
    src, dst = edge_index[0], edge_index[1]
    if e_pad > E:
        padv = jnp.full((e_pad - E,), N, dtype=src.dtype)
        src = jnp.concatenate([src, padv])
        dst = jnp.concatenate([dst, padv])

    hist = _sc_hist(dst, e_pad)
    u0, dinv = _tc_prep(x, W1, hist.T)
    u0p = jnp.pad(u0, ((0, NPAD - N), (0, 0)))
    p = _sc_scatter(u0p, src, dst, e_pad)
    u1 = _tc_mid(p[0, :N], p[1, :N], u0, dinv,
                 b1.reshape(1, HID), W2)
    u1p = jnp.pad(u1, ((0, NPAD - N), (0, 0)))
    p2 = _sc_scatter(u1p, src, dst, e_pad)
    out = _tc_final(p2[0, :N], p2[1, :N], u1, dinv, b2.reshape(1, HID),
                    W3, b3.reshape(1, -1), W4, b4.reshape(1, -1),
                    W5, b5.reshape(1, -1))
    return out


# speedup vs baseline: 18.6837x; 1.2593x over previous
import functools

import jax
import jax.numpy as jnp
from jax import lax
from jax.experimental import pallas as pl
from jax.experimental.pallas import tpu as pltpu
from jax.experimental.pallas import tpu_sc as plsc

N = 10000
HID = 64
NC, NS, L = 2, 16, 16
NW = NC * NS
NPAD = 10240
CH = 128
ROWS_PER_TILE = NPAD // NS


def _mesh():
    return plsc.VectorSubcoreMesh(core_axis_name="c", subcore_axis_name="s")


@functools.partial(jax.jit, static_argnums=(1,))
def _sc_hist(dst_pad, e_pad):
    epw = e_pad // NW

    @functools.partial(
        pl.kernel,
        out_type=jax.ShapeDtypeStruct((NW * NPAD,), jnp.float32),
        mesh=_mesh(),
        scratch_types=[
            pltpu.VMEM((NPAD,), jnp.float32),
            pltpu.VMEM((epw,), jnp.int32),
        ],
        compiler_params=pltpu.CompilerParams(needs_layout_passes=False),
    )
    def hist_k(dst_hbm, out_hbm, hist_v, idx_v):
        cid = lax.axis_index("c")
        sid = lax.axis_index("s")
        wid = cid * NS + sid

        def zbody(i, carry):
            hist_v[pl.ds(i * L, L)] = jnp.zeros((L,), jnp.float32)
            return carry

        lax.fori_loop(0, NPAD // L, zbody, 0)
        pltpu.sync_copy(dst_hbm.at[pl.ds(wid * epw, epw)], idx_v)

        ones = jnp.ones((L,), jnp.float32)

        def ibody(j, c2):
            idx = idx_v[pl.ds(j * L, L)]
            plsc.addupdate_scatter(hist_v, [idx], ones)
            return c2

        lax.fori_loop(0, epw // L, ibody, 0)
        pltpu.sync_copy(hist_v, out_hbm.at[pl.ds(wid * NPAD, NPAD)])

    return hist_k(dst_pad).reshape(NW, NPAD)


NBUF = 8
GRP = 4


@functools.partial(jax.jit, static_argnums=(3,))
def _sc_scatter(u_pad, src2d, dst2d, e_pad):
    n_chunks = e_pad // (NW * CH)

    @functools.partial(
        pl.kernel,
        out_type=jax.ShapeDtypeStruct((NC * NPAD, HID), jnp.float32),
        mesh=_mesh(),
        scratch_types=[
            pltpu.VMEM_SHARED((NPAD, HID), jnp.float32),
            pltpu.VMEM((n_chunks, CH), jnp.int32),
            pltpu.VMEM((n_chunks, CH), jnp.int32),
        ] + [pltpu.VMEM((CH, HID), jnp.float32) for _ in range(NBUF)] + [
            pltpu.SemaphoreType.DMA,
            pltpu.SemaphoreType.DMA,
        ],
        compiler_params=pltpu.CompilerParams(use_tc_tiling_on_sc=False),
    )
    def scat_k(u_hbm, src_hbm, dst_hbm, out_hbm, acc_sh, idx_s, idx_d,
               *rest):
        rows = rest[:NBUF]
        gsem, ssem = rest[NBUF], rest[NBUF + 1]
        cid = lax.axis_index("c")
        sid = lax.axis_index("s")
        wid = cid * NS + sid

        def zb(i, carry):
            rows[0][i, pl.ds(0, L)] = jnp.zeros((L,), jnp.float32)
            rows[0][i, pl.ds(L, L)] = jnp.zeros((L,), jnp.float32)
            rows[0][i, pl.ds(2 * L, L)] = jnp.zeros((L,), jnp.float32)
            rows[0][i, pl.ds(3 * L, L)] = jnp.zeros((L,), jnp.float32)
            return carry

        lax.fori_loop(0, CH, zb, 0)

        def zcopy(t, carry):
            pltpu.sync_copy(rows[0],
                            acc_sh.at[pl.ds(sid * ROWS_PER_TILE + t * CH, CH)])
            return carry

        lax.fori_loop(0, ROWS_PER_TILE // CH, zcopy, 0)

        pltpu.sync_copy(src_hbm.at[pl.ds(wid * n_chunks, n_chunks)], idx_s)
        pltpu.sync_copy(dst_hbm.at[pl.ds(wid * n_chunks, n_chunks)], idx_d)
        plsc.subcore_barrier()

        def gather(k):
            return pltpu.async_copy(u_hbm.at[idx_s.at[k]], rows[k % NBUF],
                                    gsem)

        def gather_wait(k):
            pltpu.make_async_copy(u_hbm.at[idx_s.at[k]], rows[k % NBUF],
                                  gsem).wait()

        def scat(k):
            return pltpu.async_copy(rows[k % NBUF], acc_sh.at[idx_d.at[k]],
                                    ssem, add=True)

        def scat_wait(k):
            pltpu.make_async_copy(rows[k % NBUF], acc_sh.at[idx_d.at[k]],
                                  ssem).wait()

        n_groups = n_chunks // GRP
        for j in range(GRP):
            gather(j)
        for g in range(n_groups):
            ks = [g * GRP + j for j in range(GRP)]
            for k in ks:
                gather_wait(k)
            for k in ks:
                if k + GRP < n_chunks:
                    gather(k + GRP)
            for k in ks:
                scat(k)
            for k in ks:
                scat_wait(k)

        plsc.subcore_barrier()
        pltpu.sync_copy(
            acc_sh.at[pl.ds(sid * ROWS_PER_TILE, ROWS_PER_TILE)],
            out_hbm.at[pl.ds(cid * NPAD + sid * ROWS_PER_TILE,
                             ROWS_PER_TILE)])

    return scat_k(u_pad, src2d, dst2d).reshape(NC, NPAD, HID)


def _tc_prep(x, W1, hist_t):
    def body(x_ref, w_ref, h_ref, u_ref, dinv_ref):
        deg = jnp.sum(h_ref[...], axis=1, keepdims=True) + 1.0
        dinv = lax.rsqrt(deg)[:N, :]
        h = jnp.dot(x_ref[...], w_ref[...],
                    preferred_element_type=jnp.float32)
        u_ref[...] = h * dinv
        dinv_ref[...] = dinv

    return pl.pallas_call(
        body,
        out_shape=[jax.ShapeDtypeStruct((N, HID), jnp.float32),
                   jax.ShapeDtypeStruct((N, 1), jnp.float32)],
    )(x, W1, hist_t)


def _tc_mid(p0, p1, u0, dinv, b1, W2):
    def body(p0_ref, p1_ref, u_ref, dinv_ref, b_ref, w_ref, out_ref):
        dinv = dinv_ref[...]
        h0 = dinv * (p0_ref[...] + p1_ref[...] + u_ref[...]) + b_ref[...]
        h0 = jnp.maximum(h0, 0.0)
        out_ref[...] = dinv * jnp.dot(h0, w_ref[...],
                                      preferred_element_type=jnp.float32)

    return pl.pallas_call(
        body,
        out_shape=jax.ShapeDtypeStruct((N, HID), jnp.float32),
    )(p0, p1, u0, dinv, b1, W2)


def _tc_final(p0, p1, u1, dinv, b2, W3, b3, W4, b4, W5, b5):
    def body(p0_ref, p1_ref, u_ref, dinv_ref, b2_ref, w3_ref, b3_ref,
             w4_ref, b4_ref, w5_ref, b5_ref, out_ref):
        h1 = dinv_ref[...] * (p0_ref[...] + p1_ref[...] + u_ref[...]) \
            + b2_ref[...]
        mean = jnp.mean(h1, axis=0, keepdims=True)
        mx = jnp.max(h1, axis=0, keepdims=True)
        logits = jnp.sum(h1 * mean, axis=1, keepdims=True)
        m = jnp.max(logits, axis=0, keepdims=True)
        e = jnp.exp(logits - m)
        att = e / jnp.sum(e, axis=0, keepdims=True)
        attp = jnp.sum(h1 * att, axis=0, keepdims=True)
        comb = jnp.concatenate([mean, mx, attp], axis=1)
        g = jnp.maximum(
            jnp.dot(comb, w3_ref[...], preferred_element_type=jnp.float32)
            + b3_ref[...], 0.0)
        g = jnp.maximum(
            jnp.dot(g, w4_ref[...], preferred_element_type=jnp.float32)
            + b4_ref[...], 0.0)
        out_ref[...] = jnp.dot(g, w5_ref[...],
                               preferred_element_type=jnp.float32) + b5_ref[...]

    return pl.pallas_call(
        body,
        out_shape=jax.ShapeDtypeStruct((1, 128), jnp.float32),
    )(p0, p1, u1, dinv, b2, W3, b3, W4, b4, W5, b5)


def kernel(x, edge_index, W1, b1, W2, b2, W3, b3, W4, b4, W5, b5):
    E = edge_index.shape[1]
    unit = NW * CH * GRP
    e_pad = ((E + unit - 1) // unit) * unit
    src, dst = edge_index[0], edge_index[1]
    if e_pad > E:
        padv = jnp.full((e_pad - E,), N, dtype=src.dtype)
        src = jnp.concatenate([src, padv])
        dst = jnp.concatenate([dst, padv])
    src2d = src.reshape(e_pad // CH, CH)
    dst2d = dst.reshape(e_pad // CH, CH)

    hist = _sc_hist(dst, e_pad)
    u0, dinv = _tc_prep(x, W1, hist.T)
    u0p = jnp.pad(u0, ((0, NPAD - N), (0, 0)))
    p = _sc_scatter(u0p, src2d, dst2d, e_pad)
    u1 = _tc_mid(p[0, :N], p[1, :N], u0, dinv,
                 b1.reshape(1, HID), W2)
    u1p = jnp.pad(u1, ((0, NPAD - N), (0, 0)))
    p2 = _sc_scatter(u1p, src2d, dst2d, e_pad)
    out = _tc_final(p2[0, :N], p2[1, :N], u1, dinv, b2.reshape(1, HID),
                    W3, b3.reshape(1, -1), W4, b4.reshape(1, -1),
                    W5, b5.reshape(1, -1))
    return out


# speedup vs baseline: 40.7311x; 2.1800x over previous
import functools

import jax
import jax.numpy as jnp
from jax import lax
from jax.experimental import pallas as pl
from jax.experimental.pallas import tpu as pltpu
from jax.experimental.pallas import tpu_sc as plsc

N = 10000
HID = 64
NC, NS, L = 2, 16, 16
NW = NC * NS
NPAD = 10240
CH = 128
ROWS_PER_TILE = NPAD // NS


def _mesh():
    return plsc.VectorSubcoreMesh(core_axis_name="c", subcore_axis_name="s")


@functools.partial(jax.jit, static_argnums=(1,))
def _sc_hist(dst_pad, e_pad):
    epw = e_pad // NW

    @functools.partial(
        pl.kernel,
        out_type=jax.ShapeDtypeStruct((NW * NPAD,), jnp.float32),
        mesh=_mesh(),
        scratch_types=[
            pltpu.VMEM((NPAD,), jnp.float32),
            pltpu.VMEM((epw,), jnp.int32),
        ],
        compiler_params=pltpu.CompilerParams(needs_layout_passes=False),
    )
    def hist_k(dst_hbm, out_hbm, hist_v, idx_v):
        cid = lax.axis_index("c")
        sid = lax.axis_index("s")
        wid = cid * NS + sid

        def zbody(i, carry):
            hist_v[pl.ds(i * L, L)] = jnp.zeros((L,), jnp.float32)
            return carry

        lax.fori_loop(0, NPAD // L, zbody, 0)
        pltpu.sync_copy(dst_hbm.at[pl.ds(wid * epw, epw)], idx_v)

        ones = jnp.ones((L,), jnp.float32)

        def ibody(j, c2):
            idx = idx_v[pl.ds(j * L, L)]
            plsc.addupdate_scatter(hist_v, [idx], ones)
            return c2

        lax.fori_loop(0, epw // L, ibody, 0)
        pltpu.sync_copy(hist_v, out_hbm.at[pl.ds(wid * NPAD, NPAD)])

    return hist_k(dst_pad).reshape(NW, NPAD)


NBUF = 8
GRP = 4


@functools.partial(jax.jit, static_argnums=(3,))
def _sc_scatter(u_pad, src2d, dst2d, e_pad):
    n_chunks = e_pad // (NW * CH)

    @functools.partial(
        pl.kernel,
        out_type=jax.ShapeDtypeStruct((NC * NPAD, HID), jnp.float32),
        mesh=_mesh(),
        scratch_types=[
            pltpu.VMEM_SHARED((NPAD, HID), jnp.float32),
            pltpu.VMEM((n_chunks, CH), jnp.int32),
            pltpu.VMEM((n_chunks, CH), jnp.int32),
        ] + [pltpu.VMEM((CH, HID), jnp.float32) for _ in range(NBUF)] + [
            pltpu.SemaphoreType.DMA,
            pltpu.SemaphoreType.DMA,
        ],
        compiler_params=pltpu.CompilerParams(use_tc_tiling_on_sc=False),
    )
    def scat_k(u_hbm, src_hbm, dst_hbm, out_hbm, acc_sh, idx_s, idx_d,
               *rest):
        rows = rest[:NBUF]
        gsem, ssem = rest[NBUF], rest[NBUF + 1]
        cid = lax.axis_index("c")
        sid = lax.axis_index("s")
        wid = cid * NS + sid

        def zb(i, carry):
            rows[0][i, pl.ds(0, L)] = jnp.zeros((L,), jnp.float32)
            rows[0][i, pl.ds(L, L)] = jnp.zeros((L,), jnp.float32)
            rows[0][i, pl.ds(2 * L, L)] = jnp.zeros((L,), jnp.float32)
            rows[0][i, pl.ds(3 * L, L)] = jnp.zeros((L,), jnp.float32)
            return carry

        lax.fori_loop(0, CH, zb, 0)

        def zcopy(t, carry):
            pltpu.sync_copy(rows[0],
                            acc_sh.at[pl.ds(sid * ROWS_PER_TILE + t * CH, CH)])
            return carry

        lax.fori_loop(0, ROWS_PER_TILE // CH, zcopy, 0)

        pltpu.sync_copy(src_hbm.at[pl.ds(wid * n_chunks, n_chunks)], idx_s)
        pltpu.sync_copy(dst_hbm.at[pl.ds(wid * n_chunks, n_chunks)], idx_d)
        plsc.subcore_barrier()

        def gather(k):
            return pltpu.async_copy(u_hbm.at[idx_s.at[k]], rows[k % NBUF],
                                    gsem)

        def gather_wait(k):
            pltpu.make_async_copy(u_hbm.at[idx_s.at[k]], rows[k % NBUF],
                                  gsem).wait()

        def scat(k):
            return pltpu.async_copy(rows[k % NBUF], acc_sh.at[idx_d.at[k]],
                                    ssem, add=True)

        def scat_wait(k):
            pltpu.make_async_copy(rows[k % NBUF], acc_sh.at[idx_d.at[k]],
                                  ssem).wait()

        n_groups = n_chunks // GRP
        for j in range(GRP):
            gather(j)
        for g in range(n_groups):
            ks = [g * GRP + j for j in range(GRP)]
            for k in ks:
                gather_wait(k)
            for k in ks:
                if k + GRP < n_chunks:
                    gather(k + GRP)
            for k in ks:
                scat(k)
            for k in ks:
                scat_wait(k)

        plsc.subcore_barrier()
        pltpu.sync_copy(
            acc_sh.at[pl.ds(sid * ROWS_PER_TILE, ROWS_PER_TILE)],
            out_hbm.at[pl.ds(cid * NPAD + sid * ROWS_PER_TILE,
                             ROWS_PER_TILE)])

    return scat_k(u_pad, src2d, dst2d).reshape(NC, NPAD, HID)


def _tc_prep(x, W1, hist_t):
    def body(x_ref, w_ref, h_ref, u_ref, dinv_ref):
        deg = jnp.sum(h_ref[...], axis=1, keepdims=True) + 1.0
        dinv = lax.rsqrt(deg)[:N, :]
        h = jnp.dot(x_ref[...], w_ref[...],
                    preferred_element_type=jnp.float32)
        u_ref[...] = h * dinv
        dinv_ref[...] = dinv

    return pl.pallas_call(
        body,
        out_shape=[jax.ShapeDtypeStruct((N, HID), jnp.float32),
                   jax.ShapeDtypeStruct((N, 1), jnp.float32)],
    )(x, W1, hist_t)


def _tc_mid(p0, p1, u0, dinv, b1, W2):
    def body(p0_ref, p1_ref, u_ref, dinv_ref, b_ref, w_ref, out_ref):
        dinv = dinv_ref[...]
        h0 = dinv * (p0_ref[...] + p1_ref[...] + u_ref[...]) + b_ref[...]
        h0 = jnp.maximum(h0, 0.0)
        out_ref[...] = dinv * jnp.dot(h0, w_ref[...],
                                      preferred_element_type=jnp.float32)

    return pl.pallas_call(
        body,
        out_shape=jax.ShapeDtypeStruct((N, HID), jnp.float32),
    )(p0, p1, u0, dinv, b1, W2)


def _tc_final(p0, p1, u1, dinv, b2, W3, b3, W4, b4, W5, b5):
    def body(p0_ref, p1_ref, u_ref, dinv_ref, b2_ref, w3_ref, b3_ref,
             w4_ref, b4_ref, w5_ref, b5_ref, out_ref):
        h1 = dinv_ref[...] * (p0_ref[...] + p1_ref[...] + u_ref[...]) \
            + b2_ref[...]
        mean = jnp.mean(h1, axis=0, keepdims=True)
        mx = jnp.max(h1, axis=0, keepdims=True)
        logits = jnp.sum(h1 * mean, axis=1, keepdims=True)
        m = jnp.max(logits, axis=0, keepdims=True)
        e = jnp.exp(logits - m)
        att = e / jnp.sum(e, axis=0, keepdims=True)
        attp = jnp.sum(h1 * att, axis=0, keepdims=True)
        comb = jnp.concatenate([mean, mx, attp], axis=1)
        g = jnp.maximum(
            jnp.dot(comb, w3_ref[...], preferred_element_type=jnp.float32)
            + b3_ref[...], 0.0)
        g = jnp.maximum(
            jnp.dot(g, w4_ref[...], preferred_element_type=jnp.float32)
            + b4_ref[...], 0.0)
        out_ref[...] = jnp.dot(g, w5_ref[...],
                               preferred_element_type=jnp.float32) + b5_ref[...]

    return pl.pallas_call(
        body,
        out_shape=jax.ShapeDtypeStruct((1, 128), jnp.float32),
    )(p0, p1, u1, dinv, b2, W3, b3, W4, b4, W5, b5)


def kernel(x, edge_index, W1, b1, W2, b2, W3, b3, W4, b4, W5, b5):
    E = edge_index.shape[1]
    unit = NW * CH * GRP
    e_pad = ((E + unit - 1) // unit) * unit
    src, dst = edge_index[0], edge_index[1]
    if e_pad > E:
        padv = N + (jnp.arange(e_pad - E, dtype=src.dtype) % (NPAD - N))
        src = jnp.concatenate([src, padv])
        dst = jnp.concatenate([dst, padv])
    src2d = src.reshape(e_pad // CH, CH)
    dst2d = dst.reshape(e_pad // CH, CH)

    hist = _sc_hist(dst, e_pad)
    u0, dinv = _tc_prep(x, W1, hist.T)
    u0p = jnp.pad(u0, ((0, NPAD - N), (0, 0)))
    p = _sc_scatter(u0p, src2d, dst2d, e_pad)
    u1 = _tc_mid(p[0, :N], p[1, :N], u0, dinv,
                 b1.reshape(1, HID), W2)
    u1p = jnp.pad(u1, ((0, NPAD - N), (0, 0)))
    p2 = _sc_scatter(u1p, src2d, dst2d, e_pad)
    out = _tc_final(p2[0, :N], p2[1, :N], u1, dinv, b2.reshape(1, HID),
                    W3, b3.reshape(1, -1), W4, b4.reshape(1, -1),
                    W5, b5.reshape(1, -1))
    return out
